# baseline (speedup 1.0000x reference)
import jax
import jax.numpy as jnp
from jax import lax
from jax.experimental import pallas as pl
from jax.experimental.pallas import tpu as pltpu

N = 32768
B = 128
RB = 4096
NS = 8
NCH = N // RB
GR = RB // 8
ZRB = 8192
NZ = N // ZRB
BIG = 9999999999.9
BIGF = 3.0e38


def _body(x_any, out_any, b0, b1, b2, b3, b4, b5, b6, b7, zbuf, rio,
          rm8, ri8, idxv, ohmat, idx_smem, s0, s1, s2, s3, s4, s5, s6,
          s7, sem_z, sem_s, sem_f):
    bufs = [b0, b1, b2, b3, b4, b5, b6, b7]
    sems = [s0, s1, s2, s3, s4, s5, s6, s7]

    rm8[...] = jnp.full((8, B), jnp.inf, jnp.float32)
    ri8[...] = jnp.zeros((8, B), jnp.float32)
    rio[...] = (lax.broadcasted_iota(jnp.int32, (GR, 8, B), 0) * 8
                + lax.broadcasted_iota(jnp.int32, (GR, 8, B), 1)
                ).astype(jnp.float32)

    descs = [
        pltpu.make_async_copy(
            x_any.at[pl.ds(N + ch * RB, RB), :], bufs[ch % NS],
            sems[ch % NS])
        for ch in range(NCH)
    ]
    for ch in range(NS):
        descs[ch].start()

    zbuf[...] = jnp.zeros((B, ZRB), jnp.float32)
    zdescs = [
        pltpu.make_async_copy(
            zbuf, out_any.at[:, pl.ds(z * ZRB, ZRB)], sem_z)
        for z in range(NZ)
    ]

    for ch in range(NCH):
        descs[ch].wait()
        bx = bufs[ch % NS][...].reshape(GR, 8, B)
        bz = jnp.where(bx == jnp.float32(0.0), jnp.float32(BIG), bx)
        pm = jnp.min(bz, axis=0)
        pif = jnp.min(jnp.where(bz == pm[None], rio[...],
                                jnp.float32(BIGF)), axis=0)
        pred = pm < rm8[...]
        ri8[...] = jnp.where(pred, pif + jnp.float32(ch * RB), ri8[...])
        rm8[...] = jnp.where(pred, pm, rm8[...])
        if ch % 2 == 1:
            zdescs[ch // 2].start()

    m = jnp.min(rm8[...], axis=0, keepdims=True)
    idxf = jnp.min(jnp.where(rm8[...] == m, ri8[...],
                             jnp.float32(BIGF)),
                   axis=0, keepdims=True)
    idxv[...] = idxf.astype(jnp.int32)

    eye = jnp.where(
        lax.broadcasted_iota(jnp.int32, (B, B), 0)
        == lax.broadcasted_iota(jnp.int32, (B, B), 1),
        jnp.float32(1.0), jnp.float32(0.0))
    col = lax.dot_general(eye, idxf, (((1,), (1,)), ((), ())),
                          preferred_element_type=jnp.float32)
    base = jnp.floor(col * jnp.float32(1.0 / B)) * jnp.float32(B)
    mod = jnp.broadcast_to(col - base, (B, B))
    ohmat[...] = jnp.where(
        lax.broadcasted_iota(jnp.int32, (B, B), 1).astype(jnp.float32)
        == mod, jnp.float32(1.0), jnp.float32(0.0))

    pltpu.make_async_copy(idxv, idx_smem, sem_s).start()

    for z in range(NZ):
        zdescs[z].wait()
    pltpu.make_async_copy(idxv, idx_smem, sem_s).wait()

    fdescs = []
    for j in range(B):
        bj = idx_smem[0, j]
        cbase = (bj // B) * B
        d = pltpu.make_async_copy(
            ohmat.at[j], out_any.at[j, pl.ds(cbase, B)], sem_f)
        d.start()
        fdescs.append(d)
    for d in fdescs:
        d.wait()


@jax.jit
def kernel(x):
    return pl.pallas_call(
        _body,
        out_shape=jax.ShapeDtypeStruct((B, N), jnp.float32),
        grid=(1,),
        in_specs=[pl.BlockSpec(memory_space=pl.ANY)],
        out_specs=pl.BlockSpec(memory_space=pl.ANY),
        scratch_shapes=[
            pltpu.VMEM((RB, B), jnp.float32),
            pltpu.VMEM((RB, B), jnp.float32),
            pltpu.VMEM((RB, B), jnp.float32),
            pltpu.VMEM((RB, B), jnp.float32),
            pltpu.VMEM((RB, B), jnp.float32),
            pltpu.VMEM((RB, B), jnp.float32),
            pltpu.VMEM((RB, B), jnp.float32),
            pltpu.VMEM((RB, B), jnp.float32),
            pltpu.VMEM((B, ZRB), jnp.float32),
            pltpu.VMEM((GR, 8, B), jnp.float32),
            pltpu.VMEM((8, B), jnp.float32),
            pltpu.VMEM((8, B), jnp.float32),
            pltpu.VMEM((1, B), jnp.int32),
            pltpu.VMEM((B, B), jnp.float32),
            pltpu.SMEM((1, B), jnp.int32),
            pltpu.SemaphoreType.DMA,
            pltpu.SemaphoreType.DMA,
            pltpu.SemaphoreType.DMA,
            pltpu.SemaphoreType.DMA,
            pltpu.SemaphoreType.DMA,
            pltpu.SemaphoreType.DMA,
            pltpu.SemaphoreType.DMA,
            pltpu.SemaphoreType.DMA,
            pltpu.SemaphoreType.DMA,
            pltpu.SemaphoreType.DMA,
            pltpu.SemaphoreType.DMA,
        ],
    )(x)


# speedup vs baseline: 1.0040x; 1.0040x over previous
import jax
import jax.numpy as jnp
from jax import lax
from jax.experimental import pallas as pl
from jax.experimental.pallas import tpu as pltpu

N = 32768
B = 128
RB = 4096
NS = 4
NCH = N // RB
GR = RB // 8
ZRB = 8192
NZ = N // ZRB
BIG = 9999999999.9
BIGF = 3.0e38


def _body(x_any, out_any, b0, b1, b2, b3, zbuf, rio, rm8, ri8, idxv,
          ohmat, idx_smem, s0, s1, s2, s3, sem_z, sem_s, sem_f):
    bufs = [b0, b1, b2, b3]
    sems = [s0, s1, s2, s3]

    rm8[...] = jnp.full((8, B), jnp.inf, jnp.float32)
    ri8[...] = jnp.zeros((8, B), jnp.float32)
    rio[...] = (lax.broadcasted_iota(jnp.int32, (GR, 8, B), 0) * 8
                + lax.broadcasted_iota(jnp.int32, (GR, 8, B), 1)
                ).astype(jnp.float32)

    descs = [
        pltpu.make_async_copy(
            x_any.at[pl.ds(N + ch * RB, RB), :], bufs[ch % NS],
            sems[ch % NS])
        for ch in range(NCH)
    ]
    for ch in range(NS):
        descs[ch].start()

    zbuf[...] = jnp.zeros((B, ZRB), jnp.float32)
    zdescs = [
        pltpu.make_async_copy(
            zbuf, out_any.at[:, pl.ds(z * ZRB, ZRB)], sem_z)
        for z in range(NZ)
    ]

    for ch in range(NCH):
        descs[ch].wait()
        bx = bufs[ch % NS][...].reshape(GR, 8, B)
        bz = jnp.where(bx == jnp.float32(0.0), jnp.float32(BIG), bx)
        pm = jnp.min(bz, axis=0)
        pif = jnp.min(jnp.where(bz == pm[None], rio[...],
                                jnp.float32(BIGF)), axis=0)
        pred = pm < rm8[...]
        ri8[...] = jnp.where(pred, pif + jnp.float32(ch * RB), ri8[...])
        rm8[...] = jnp.where(pred, pm, rm8[...])
        if ch + NS < NCH:
            descs[ch + NS].start()
        if ch % 2 == 1:
            zdescs[ch // 2].start()

    m = jnp.min(rm8[...], axis=0, keepdims=True)
    idxf = jnp.min(jnp.where(rm8[...] == m, ri8[...],
                             jnp.float32(BIGF)),
                   axis=0, keepdims=True)
    idxv[...] = idxf.astype(jnp.int32)

    eye = jnp.where(
        lax.broadcasted_iota(jnp.int32, (B, B), 0)
        == lax.broadcasted_iota(jnp.int32, (B, B), 1),
        jnp.float32(1.0), jnp.float32(0.0))
    col = lax.dot_general(eye, idxf, (((1,), (1,)), ((), ())),
                          preferred_element_type=jnp.float32)
    base = jnp.floor(col * jnp.float32(1.0 / B)) * jnp.float32(B)
    mod = jnp.broadcast_to(col - base, (B, B))
    ohmat[...] = jnp.where(
        lax.broadcasted_iota(jnp.int32, (B, B), 1).astype(jnp.float32)
        == mod, jnp.float32(1.0), jnp.float32(0.0))

    pltpu.make_async_copy(idxv, idx_smem, sem_s).start()

    for z in range(NZ):
        zdescs[z].wait()
    pltpu.make_async_copy(idxv, idx_smem, sem_s).wait()

    fdescs = []
    for j in range(B):
        bj = idx_smem[0, j]
        cbase = (bj // B) * B
        d = pltpu.make_async_copy(
            ohmat.at[j], out_any.at[j, pl.ds(cbase, B)], sem_f)
        d.start()
        fdescs.append(d)
    for d in fdescs:
        d.wait()


@jax.jit
def kernel(x):
    return pl.pallas_call(
        _body,
        out_shape=jax.ShapeDtypeStruct((B, N), jnp.float32),
        grid=(1,),
        in_specs=[pl.BlockSpec(memory_space=pl.ANY)],
        out_specs=pl.BlockSpec(memory_space=pl.ANY),
        scratch_shapes=[
            pltpu.VMEM((RB, B), jnp.float32),
            pltpu.VMEM((RB, B), jnp.float32),
            pltpu.VMEM((RB, B), jnp.float32),
            pltpu.VMEM((RB, B), jnp.float32),
            pltpu.VMEM((B, ZRB), jnp.float32),
            pltpu.VMEM((GR, 8, B), jnp.float32),
            pltpu.VMEM((8, B), jnp.float32),
            pltpu.VMEM((8, B), jnp.float32),
            pltpu.VMEM((1, B), jnp.int32),
            pltpu.VMEM((B, B), jnp.float32),
            pltpu.SMEM((1, B), jnp.int32),
            pltpu.SemaphoreType.DMA,
            pltpu.SemaphoreType.DMA,
            pltpu.SemaphoreType.DMA,
            pltpu.SemaphoreType.DMA,
            pltpu.SemaphoreType.DMA,
            pltpu.SemaphoreType.DMA,
            pltpu.SemaphoreType.DMA,
        ],
    )(x)
